# Initial kernel scaffold; baseline (speedup 1.0000x reference)
import functools

import jax
import jax.numpy as jnp
from jax import lax
from jax.experimental import pallas as pl
from jax.experimental.pallas import tpu as pltpu
from jax.experimental.pallas import tpu_sc as plsc

N = 10000
E = 320000
NC = 2
NS = 16
TPR = N // NS
PER_TILE = E // (NC * NS)
CHUNK = 80
NCHUNK = PER_TILE // CHUNK

_MESH = plsc.VectorSubcoreMesh(
    core_axis_name="c", subcore_axis_name="s", num_cores=NC, num_subcores=NS)


def _sc_agg(table, src, dst, zeros):
  D = table.shape[1]

  @functools.partial(
      pl.kernel,
      out_type=jax.ShapeDtypeStruct((NC, N, D), jnp.float32),
      mesh=_MESH,
      scratch_types=[
          pltpu.VMEM((CHUNK,), jnp.int32),
          pltpu.VMEM((CHUNK,), jnp.int32),
          pltpu.VMEM((CHUNK, D), jnp.float32),
          pltpu.VMEM_SHARED((N, D), jnp.float32),
          pltpu.SemaphoreType.DMA,
      ],
  )
  def k(table_hbm, src_hbm, dst_hbm, zeros_hbm, out_hbm,
        idx_s, idx_d, rows, acc, sem):
    ci = lax.axis_index("c")
    si = lax.axis_index("s")
    pltpu.sync_copy(zeros_hbm, acc.at[pl.ds(si * TPR, TPR)])
    plsc.subcore_barrier()
    base = ci * (E // NC) + si * PER_TILE

    def body(kk, carry):
      off = base + kk * CHUNK
      pltpu.sync_copy(src_hbm.at[pl.ds(off, CHUNK)], idx_s)
      pltpu.sync_copy(dst_hbm.at[pl.ds(off, CHUNK)], idx_d)
      pltpu.async_copy(table_hbm.at[idx_s], rows, sem).wait()
      pltpu.sync_copy(rows, acc.at[idx_d], add=True)
      return carry

    lax.fori_loop(0, NCHUNK, body, 0)
    plsc.subcore_barrier()
    pltpu.sync_copy(acc.at[pl.ds(si * TPR, TPR)],
                    out_hbm.at[ci, pl.ds(si * TPR, TPR)])

  return k(table, src, dst, zeros)


def _sc_deg(dst, ones, zeros):
  D = ones.shape[1]

  @functools.partial(
      pl.kernel,
      out_type=jax.ShapeDtypeStruct((NC, N, D), jnp.float32),
      mesh=_MESH,
      scratch_types=[
          pltpu.VMEM((CHUNK,), jnp.int32),
          pltpu.VMEM((CHUNK, D), jnp.float32),
          pltpu.VMEM_SHARED((N, D), jnp.float32),
          pltpu.SemaphoreType.DMA,
      ],
  )
  def k(dst_hbm, ones_hbm, zeros_hbm, out_hbm, idx_d, rows, acc, sem):
    ci = lax.axis_index("c")
    si = lax.axis_index("s")
    pltpu.sync_copy(zeros_hbm, acc.at[pl.ds(si * TPR, TPR)])
    pltpu.sync_copy(ones_hbm, rows)
    plsc.subcore_barrier()
    base = ci * (E // NC) + si * PER_TILE

    def body(kk, carry):
      off = base + kk * CHUNK
      pltpu.sync_copy(dst_hbm.at[pl.ds(off, CHUNK)], idx_d)
      pltpu.sync_copy(rows, acc.at[idx_d], add=True)
      return carry

    lax.fori_loop(0, NCHUNK, body, 0)
    plsc.subcore_barrier()
    pltpu.sync_copy(acc.at[pl.ds(si * TPR, TPR)],
                    out_hbm.at[ci, pl.ds(si * TPR, TPR)])

  return k(dst, ones, zeros)


_R = 1000


def _tc_t1(x, W1, degp):
  def body(x_ref, w_ref, dp_ref, t1_ref, deg_ref):
    deg = dp_ref[0] + dp_ref[1] + 1.0
    dinv = lax.rsqrt(deg[:, :1])
    t1_ref[...] = dinv * jnp.dot(x_ref[...], w_ref[...],
                                 preferred_element_type=jnp.float32)
    deg_ref[...] = deg

  return pl.pallas_call(
      body,
      grid=(N // _R,),
      in_specs=[
          pl.BlockSpec((_R, x.shape[1]), lambda i: (i, 0)),
          pl.BlockSpec(W1.shape, lambda i: (0, 0)),
          pl.BlockSpec((NC, _R, degp.shape[2]), lambda i: (0, i, 0)),
      ],
      out_specs=[
          pl.BlockSpec((_R, W1.shape[1]), lambda i: (i, 0)),
          pl.BlockSpec((_R, degp.shape[2]), lambda i: (i, 0)),
      ],
      out_shape=[
          jax.ShapeDtypeStruct((N, W1.shape[1]), jnp.float32),
          jax.ShapeDtypeStruct((N, degp.shape[2]), jnp.float32),
      ],
  )(x, W1, degp)


def _tc_t2(accp, T1, deg, b1, W2):
  def body(a_ref, t1_ref, deg_ref, b_ref, w_ref, t2_ref):
    dinv = lax.rsqrt(deg_ref[:, :1])
    h = jnp.maximum(dinv * (a_ref[0] + a_ref[1] + t1_ref[...]) + b_ref[...],
                    0.0)
    t2_ref[...] = dinv * jnp.dot(h, w_ref[...],
                                 preferred_element_type=jnp.float32)

  D = T1.shape[1]
  return pl.pallas_call(
      body,
      grid=(N // _R,),
      in_specs=[
          pl.BlockSpec((NC, _R, D), lambda i: (0, i, 0)),
          pl.BlockSpec((_R, D), lambda i: (i, 0)),
          pl.BlockSpec((_R, deg.shape[1]), lambda i: (i, 0)),
          pl.BlockSpec(b1.shape, lambda i: (0, 0)),
          pl.BlockSpec(W2.shape, lambda i: (0, 0)),
      ],
      out_specs=pl.BlockSpec((_R, W2.shape[1]), lambda i: (i, 0)),
      out_shape=jax.ShapeDtypeStruct((N, W2.shape[1]), jnp.float32),
  )(accp, T1, deg, b1, W2)


def _tc_out(accp, T2, deg, b2):
  def body(a_ref, t2_ref, deg_ref, b_ref, o_ref):
    dinv = lax.rsqrt(deg_ref[:, :1])
    o_ref[...] = dinv * (a_ref[0] + a_ref[1] + t2_ref[...]) + b_ref[...]

  D = T2.shape[1]
  return pl.pallas_call(
      body,
      grid=(N // _R,),
      in_specs=[
          pl.BlockSpec((NC, _R, D), lambda i: (0, i, 0)),
          pl.BlockSpec((_R, D), lambda i: (i, 0)),
          pl.BlockSpec((_R, deg.shape[1]), lambda i: (i, 0)),
          pl.BlockSpec(b2.shape, lambda i: (0, 0)),
      ],
      out_specs=pl.BlockSpec((_R, D), lambda i: (i, 0)),
      out_shape=jax.ShapeDtypeStruct((N, D), jnp.float32),
  )(accp, T2, deg, b2)


def kernel(x, edge_index, W1, b1, W2, b2):
  src = edge_index[0]
  dst = edge_index[1]
  ones8 = jnp.ones((CHUNK, 8), jnp.float32)
  z8 = jnp.zeros((TPR, 8), jnp.float32)
  z64 = jnp.zeros((TPR, 64), jnp.float32)
  z128 = jnp.zeros((TPR, 128), jnp.float32)

  degp = _sc_deg(dst, ones8, z8)
  T1, deg = _tc_t1(x, W1, degp)
  acc1 = _sc_agg(T1, src, dst, z64)
  T2 = _tc_t2(acc1, T1, deg, b1.reshape(1, -1), W2)
  acc2 = _sc_agg(T2, src, dst, z128)
  return _tc_out(acc2, T2, deg, b2.reshape(1, -1))


# speedup vs baseline: 14.0983x; 14.0983x over previous
import functools

import jax
import jax.numpy as jnp
from jax import lax
from jax.experimental import pallas as pl
from jax.experimental.pallas import tpu as pltpu
from jax.experimental.pallas import tpu_sc as plsc

N = 10000
NP = 10240
E = 320000
NC = 2
NS = 16
TPR = NP // NS
PER_TILE = E // (NC * NS)
CHUNK = 80
NCHUNK = PER_TILE // CHUNK

_MESH = plsc.VectorSubcoreMesh(
    core_axis_name="c", subcore_axis_name="s", num_cores=NC, num_subcores=NS)


def _sc_agg(table, src, dst, zeros):
  D = table.shape[1]

  @functools.partial(
      pl.kernel,
      out_type=jax.ShapeDtypeStruct((NC, NP, D), jnp.float32),
      mesh=_MESH,
      compiler_params=pltpu.CompilerParams(use_tc_tiling_on_sc=False),
      scratch_types=[
          pltpu.VMEM((CHUNK,), jnp.int32),
          pltpu.VMEM((CHUNK,), jnp.int32),
          pltpu.VMEM((CHUNK, D), jnp.float32),
          pltpu.VMEM_SHARED((NP, D), jnp.float32),
          pltpu.SemaphoreType.DMA,
      ],
  )
  def k(table_hbm, src_hbm, dst_hbm, zeros_hbm, out_hbm,
        idx_s, idx_d, rows, acc, sem):
    ci = lax.axis_index("c")
    si = lax.axis_index("s")
    pltpu.sync_copy(zeros_hbm, acc.at[pl.ds(si * TPR, TPR)])
    plsc.subcore_barrier()
    base = ci * (E // NC) + si * PER_TILE

    def body(kk, carry):
      off = base + kk * CHUNK
      pltpu.sync_copy(src_hbm.at[pl.ds(off, CHUNK)], idx_s)
      pltpu.sync_copy(dst_hbm.at[pl.ds(off, CHUNK)], idx_d)
      pltpu.async_copy(table_hbm.at[idx_s], rows, sem).wait()
      pltpu.sync_copy(rows, acc.at[idx_d], add=True)
      return carry

    lax.fori_loop(0, NCHUNK, body, 0)
    plsc.subcore_barrier()
    pltpu.sync_copy(acc.at[pl.ds(si * TPR, TPR)],
                    out_hbm.at[ci, pl.ds(si * TPR, TPR)])

  return k(table, src, dst, zeros)


def _sc_deg(dst, ones, zeros):
  D = ones.shape[1]

  @functools.partial(
      pl.kernel,
      out_type=jax.ShapeDtypeStruct((NC, NP, D), jnp.float32),
      mesh=_MESH,
      compiler_params=pltpu.CompilerParams(use_tc_tiling_on_sc=False),
      scratch_types=[
          pltpu.VMEM((CHUNK,), jnp.int32),
          pltpu.VMEM((CHUNK, D), jnp.float32),
          pltpu.VMEM_SHARED((NP, D), jnp.float32),
          pltpu.SemaphoreType.DMA,
      ],
  )
  def k(dst_hbm, ones_hbm, zeros_hbm, out_hbm, idx_d, rows, acc, sem):
    ci = lax.axis_index("c")
    si = lax.axis_index("s")
    pltpu.sync_copy(zeros_hbm, acc.at[pl.ds(si * TPR, TPR)])
    pltpu.sync_copy(ones_hbm, rows)
    plsc.subcore_barrier()
    base = ci * (E // NC) + si * PER_TILE

    def body(kk, carry):
      off = base + kk * CHUNK
      pltpu.sync_copy(dst_hbm.at[pl.ds(off, CHUNK)], idx_d)
      pltpu.sync_copy(rows, acc.at[idx_d], add=True)
      return carry

    lax.fori_loop(0, NCHUNK, body, 0)
    plsc.subcore_barrier()
    pltpu.sync_copy(acc.at[pl.ds(si * TPR, TPR)],
                    out_hbm.at[ci, pl.ds(si * TPR, TPR)])

  return k(dst, ones, zeros)


_R = 1024


def _tc_t1(x, W1, degp):
  def body(x_ref, w_ref, dp_ref, t1_ref, deg_ref):
    deg = dp_ref[0] + dp_ref[1] + 1.0
    dinv = lax.rsqrt(deg[:, :1])
    t1_ref[...] = dinv * jnp.dot(x_ref[...], w_ref[...],
                                 preferred_element_type=jnp.float32)
    deg_ref[...] = deg

  return pl.pallas_call(
      body,
      grid=(NP // _R,),
      in_specs=[
          pl.BlockSpec((_R, x.shape[1]), lambda i: (i, 0)),
          pl.BlockSpec(W1.shape, lambda i: (0, 0)),
          pl.BlockSpec((NC, _R, degp.shape[2]), lambda i: (0, i, 0)),
      ],
      out_specs=[
          pl.BlockSpec((_R, W1.shape[1]), lambda i: (i, 0)),
          pl.BlockSpec((_R, degp.shape[2]), lambda i: (i, 0)),
      ],
      out_shape=[
          jax.ShapeDtypeStruct((NP, W1.shape[1]), jnp.float32),
          jax.ShapeDtypeStruct((NP, degp.shape[2]), jnp.float32),
      ],
  )(x, W1, degp)


def _tc_t2(accp, T1, deg, b1, W2):
  def body(a_ref, t1_ref, deg_ref, b_ref, w_ref, t2_ref):
    dinv = lax.rsqrt(deg_ref[:, :1])
    h = jnp.maximum(dinv * (a_ref[0] + a_ref[1] + t1_ref[...]) + b_ref[...],
                    0.0)
    t2_ref[...] = dinv * jnp.dot(h, w_ref[...],
                                 preferred_element_type=jnp.float32)

  D = T1.shape[1]
  return pl.pallas_call(
      body,
      grid=(NP // _R,),
      in_specs=[
          pl.BlockSpec((NC, _R, D), lambda i: (0, i, 0)),
          pl.BlockSpec((_R, D), lambda i: (i, 0)),
          pl.BlockSpec((_R, deg.shape[1]), lambda i: (i, 0)),
          pl.BlockSpec(b1.shape, lambda i: (0, 0)),
          pl.BlockSpec(W2.shape, lambda i: (0, 0)),
      ],
      out_specs=pl.BlockSpec((_R, W2.shape[1]), lambda i: (i, 0)),
      out_shape=jax.ShapeDtypeStruct((NP, W2.shape[1]), jnp.float32),
  )(accp, T1, deg, b1, W2)


def _tc_out(accp, T2, deg, b2):
  def body(a_ref, t2_ref, deg_ref, b_ref, o_ref):
    dinv = lax.rsqrt(deg_ref[:, :1])
    o_ref[...] = dinv * (a_ref[0] + a_ref[1] + t2_ref[...]) + b_ref[...]

  D = T2.shape[1]
  return pl.pallas_call(
      body,
      grid=(NP // _R,),
      in_specs=[
          pl.BlockSpec((NC, _R, D), lambda i: (0, i, 0)),
          pl.BlockSpec((_R, D), lambda i: (i, 0)),
          pl.BlockSpec((_R, deg.shape[1]), lambda i: (i, 0)),
          pl.BlockSpec(b2.shape, lambda i: (0, 0)),
      ],
      out_specs=pl.BlockSpec((_R, D), lambda i: (i, 0)),
      out_shape=jax.ShapeDtypeStruct((NP, D), jnp.float32),
  )(accp, T2, deg, b2)


def kernel(x, edge_index, W1, b1, W2, b2):
  src = edge_index[0]
  dst = edge_index[1]
  x = jnp.pad(x, ((0, NP - N), (0, 0)))
  ones8 = jnp.ones((CHUNK, 8), jnp.float32)
  z8 = jnp.zeros((TPR, 8), jnp.float32)
  z64 = jnp.zeros((TPR, 64), jnp.float32)
  z128 = jnp.zeros((TPR, 128), jnp.float32)

  degp = _sc_deg(dst, ones8, z8)
  T1, deg = _tc_t1(x, W1, degp)
  acc1 = _sc_agg(T1, src, dst, z64)
  T2 = _tc_t2(acc1, T1, deg, b1.reshape(1, -1), W2)
  acc2 = _sc_agg(T2, src, dst, z128)
  out = _tc_out(acc2, T2, deg, b2.reshape(1, -1))
  return out[:N]


# speedup vs baseline: 15.2500x; 1.0817x over previous
import functools

import jax
import jax.numpy as jnp
from jax import lax
from jax.experimental import pallas as pl
from jax.experimental.pallas import tpu as pltpu
from jax.experimental.pallas import tpu_sc as plsc

N = 10000
NP = 10240
E = 320000
NC = 2
NS = 16
TPR = NP // NS
PER_TILE = 10240
E_PAD = NC * NS * PER_TILE
CH = 128

_MESH = plsc.VectorSubcoreMesh(
    core_axis_name="c", subcore_axis_name="s", num_cores=NC, num_subcores=NS)


def _sc_agg(table0, table1, src2d, dst2d, zeros, per_tile, core_stride):
  D = table0.shape[1]
  nchunk = per_tile // CH
  ng = nchunk // 8

  @functools.partial(
      pl.kernel,
      out_type=jax.ShapeDtypeStruct((NC, NP, D), jnp.float32),
      mesh=_MESH,
      compiler_params=pltpu.CompilerParams(use_tc_tiling_on_sc=False),
      scratch_types=(
          [pltpu.VMEM((CH,), jnp.int32) for _ in range(16)]
          + [pltpu.VMEM((CH, D), jnp.float32) for _ in range(8)]
          + [pltpu.VMEM_SHARED((NP, D), jnp.float32)]
          + [pltpu.SemaphoreType.DMA for _ in range(28)]
      ),
  )
  def k(t0_hbm, t1_hbm, src_hbm, dst_hbm, zeros_hbm, out_hbm, *rest):
    srcb = rest[0:8]
    dstb = rest[8:16]
    rows = rest[16:24]
    acc = rest[24]
    sis = rest[25:33]
    sid = rest[33:41]
    sg = rest[41:49]
    ssc = rest[49:53]
    ci = lax.axis_index("c")
    si = lax.axis_index("s")
    pltpu.sync_copy(zeros_hbm, acc.at[pl.ds(si * TPR, TPR)])
    plsc.subcore_barrier()

    def emit(table_hbm, cbase):
      def src_dma(kk, b):
        pltpu.async_copy(src_hbm.at[cbase + kk], srcb[b], sis[b])

      def dst_dma(kk, b):
        pltpu.async_copy(dst_hbm.at[cbase + kk], dstb[b], sid[b])

      def gather(kk_b):
        pltpu.make_async_copy(src_hbm.at[0], srcb[kk_b], sis[kk_b]).wait()
        pltpu.async_copy(table_hbm.at[srcb[kk_b]], rows[kk_b], sg[kk_b])

      def scatter_wait(b4):
        pltpu.make_async_copy(rows[0], acc.at[dstb[0]], ssc[b4]).wait()

      for b in range(8):
        src_dma(b, b)
      for b in range(4):
        dst_dma(b, b)
      for b in range(4):
        gather(b)

      def body(g, carry):
        for b in range(8):
          kk = g * 8 + b
          b4 = b % 4
          bn = (b + 4) % 8
          pltpu.make_async_copy(table_hbm.at[srcb[b]], rows[b], sg[b]).wait()
          pltpu.make_async_copy(src_hbm.at[0], dstb[b], sid[b]).wait()
          if b < 4:
            pl.when(g > 0)(lambda: scatter_wait(b4))
          else:
            scatter_wait(b4)
          pltpu.async_copy(rows[b], acc.at[dstb[b]], ssc[b4], add=True)

          def refill(kk=kk, b=b, bn=bn):
            dst_dma(kk + 4, bn)
            gather(bn)
          if b < 4:
            refill()
          else:
            pl.when(g < ng - 1)(refill)
          pl.when(g < ng - 1)(lambda kk=kk, b=b: src_dma(kk + 8, b))
        return carry

      lax.fori_loop(0, ng, body, 0)
      for i in range(4):
        scatter_wait(i)

    base0 = si * (per_tile // CH)
    base1 = core_stride // CH + base0

    @pl.when(ci == 0)
    def _():
      emit(t0_hbm, base0)

    @pl.when(ci == 1)
    def _():
      emit(t1_hbm, base1)

    plsc.subcore_barrier()
    pltpu.sync_copy(acc.at[pl.ds(si * TPR, TPR)],
                    out_hbm.at[ci, pl.ds(si * TPR, TPR)])

  return k(table0, table1, src2d, dst2d, zeros)


def _sc_deg(dst2d, ones, zeros):
  D = ones.shape[1]
  chunk = dst2d.shape[1]
  nchunk = PER_TILE // chunk

  @functools.partial(
      pl.kernel,
      out_type=jax.ShapeDtypeStruct((NC, NP, D), jnp.float32),
      mesh=_MESH,
      compiler_params=pltpu.CompilerParams(use_tc_tiling_on_sc=False),
      scratch_types=(
          [pltpu.VMEM((nchunk, chunk), jnp.int32),
           pltpu.VMEM((chunk, D), jnp.float32),
           pltpu.VMEM_SHARED((NP, D), jnp.float32)]
          + [pltpu.SemaphoreType.DMA for _ in range(5)]
      ),
  )
  def k(dst_hbm, ones_hbm, zeros_hbm, out_hbm, dstb, onev, acc, *ss):
    ci = lax.axis_index("c")
    si = lax.axis_index("s")
    cbase = (ci * NS + si) * nchunk
    pltpu.sync_copy(zeros_hbm, acc.at[pl.ds(si * TPR, TPR)])
    pltpu.sync_copy(ones_hbm, onev)
    pltpu.sync_copy(dst_hbm.at[pl.ds(cbase, nchunk)], dstb)
    plsc.subcore_barrier()

    def scatter(kk, b5):
      pltpu.async_copy(onev, acc.at[dstb.at[kk]], ss[b5], add=True)

    def scatter_wait(kk, b5):
      pltpu.make_async_copy(onev, acc.at[dstb.at[kk]], ss[b5]).wait()

    for b in range(5):
      scatter(b, b)

    def body(g, carry):
      for b in range(5):
        kk = 5 + g * 5 + b
        scatter_wait(kk - 5, b)
        scatter(kk, b)
      return carry

    lax.fori_loop(0, (nchunk - 5) // 5, body, 0)
    for i in range(5):
      kk = nchunk - 5 + i
      scatter_wait(kk, i)
    plsc.subcore_barrier()
    pltpu.sync_copy(acc.at[pl.ds(si * TPR, TPR)],
                    out_hbm.at[ci, pl.ds(si * TPR, TPR)])

  return k(dst2d, ones, zeros)


_R = 1024


def _tc_t1(x, W1, degp):
  def body(x_ref, w_ref, dp_ref, t1_ref, deg_ref):
    deg = dp_ref[0] + dp_ref[1] + 1.0
    dinv = lax.rsqrt(deg[:, :1])
    t1_ref[...] = dinv * jnp.dot(x_ref[...], w_ref[...],
                                 preferred_element_type=jnp.float32)
    deg_ref[...] = deg

  return pl.pallas_call(
      body,
      grid=(NP // _R,),
      in_specs=[
          pl.BlockSpec((_R, x.shape[1]), lambda i: (i, 0)),
          pl.BlockSpec(W1.shape, lambda i: (0, 0)),
          pl.BlockSpec((NC, _R, degp.shape[2]), lambda i: (0, i, 0)),
      ],
      out_specs=[
          pl.BlockSpec((_R, W1.shape[1]), lambda i: (i, 0)),
          pl.BlockSpec((_R, degp.shape[2]), lambda i: (i, 0)),
      ],
      out_shape=[
          jax.ShapeDtypeStruct((NP, W1.shape[1]), jnp.float32),
          jax.ShapeDtypeStruct((NP, degp.shape[2]), jnp.float32),
      ],
  )(x, W1, degp)


def _tc_t2(accp, T1, deg, b1, W2):
  def body(a_ref, t1_ref, deg_ref, b_ref, w_ref, t2a_ref, t2b_ref):
    dinv = lax.rsqrt(deg_ref[:, :1])
    h = jnp.maximum(dinv * (a_ref[0] + a_ref[1] + t1_ref[...]) + b_ref[...],
                    0.0)
    t2 = dinv * jnp.dot(h, w_ref[...], preferred_element_type=jnp.float32)
    half = w_ref.shape[1] // 2
    t2a_ref[...] = t2[:, :half]
    t2b_ref[...] = t2[:, half:]

  D = T1.shape[1]
  half = W2.shape[1] // 2
  return pl.pallas_call(
      body,
      grid=(NP // _R,),
      in_specs=[
          pl.BlockSpec((NC, _R, D), lambda i: (0, i, 0)),
          pl.BlockSpec((_R, D), lambda i: (i, 0)),
          pl.BlockSpec((_R, deg.shape[1]), lambda i: (i, 0)),
          pl.BlockSpec(b1.shape, lambda i: (0, 0)),
          pl.BlockSpec(W2.shape, lambda i: (0, 0)),
      ],
      out_specs=[
          pl.BlockSpec((_R, half), lambda i: (i, 0)),
          pl.BlockSpec((_R, half), lambda i: (i, 0)),
      ],
      out_shape=[
          jax.ShapeDtypeStruct((NP, half), jnp.float32),
          jax.ShapeDtypeStruct((NP, half), jnp.float32),
      ],
  )(accp, T1, deg, b1, W2)


def _tc_out(accp, T2a, T2b, deg, b2):
  def body(a_ref, ta_ref, tb_ref, deg_ref, b_ref, o_ref):
    dinv = lax.rsqrt(deg_ref[:, :1])
    t = jnp.concatenate([a_ref[0] + ta_ref[...], a_ref[1] + tb_ref[...]],
                        axis=1)
    o_ref[...] = dinv * t + b_ref[...]

  half = T2a.shape[1]
  return pl.pallas_call(
      body,
      grid=(NP // _R,),
      in_specs=[
          pl.BlockSpec((NC, _R, half), lambda i: (0, i, 0)),
          pl.BlockSpec((_R, half), lambda i: (i, 0)),
          pl.BlockSpec((_R, half), lambda i: (i, 0)),
          pl.BlockSpec((_R, deg.shape[1]), lambda i: (i, 0)),
          pl.BlockSpec(b2.shape, lambda i: (0, 0)),
      ],
      out_specs=pl.BlockSpec((_R, 2 * half), lambda i: (i, 0)),
      out_shape=jax.ShapeDtypeStruct((NP, 2 * half), jnp.float32),
  )(accp, T2a, T2b, deg, b2)


def kernel(x, edge_index, W1, b1, W2, b2):
  pad = jnp.full((E_PAD - E,), N, jnp.int32)
  src128 = jnp.concatenate([edge_index[0], pad]).reshape(-1, CH)
  dst128 = jnp.concatenate([edge_index[1], pad]).reshape(-1, CH)
  x = jnp.pad(x, ((0, NP - N), (0, 0)))
  ones8 = jnp.ones((CH, 8), jnp.float32)
  z8 = jnp.zeros((TPR, 8), jnp.float32)
  z64 = jnp.zeros((TPR, 64), jnp.float32)

  degp = _sc_deg(dst128, ones8, z8)
  T1, deg = _tc_t1(x, W1, degp)
  acc1 = _sc_agg(T1, T1, src128, dst128, z64, PER_TILE, E_PAD // NC)
  T2a, T2b = _tc_t2(acc1, T1, deg, b1.reshape(1, -1), W2)
  acc2 = _sc_agg(T2a, T2b, src128, dst128, z64, E_PAD // NS, 0)
  out = _tc_out(acc2, T2a, T2b, deg, b2.reshape(1, -1))
  return out[:N]


# speedup vs baseline: 31.6749x; 2.0770x over previous
import functools

import jax
import jax.numpy as jnp
from jax import lax
from jax.experimental import pallas as pl
from jax.experimental.pallas import tpu as pltpu
from jax.experimental.pallas import tpu_sc as plsc

N = 10000
NP = 10240
E = 320000
NC = 2
NS = 16
TPR = NP // NS
PER_TILE = 10240
E_PAD = NC * NS * PER_TILE
CH = 64

_MESH = plsc.VectorSubcoreMesh(
    core_axis_name="c", subcore_axis_name="s", num_cores=NC, num_subcores=NS)


def _sc_agg(table0, table1, src2d, dst2d, zeros, per_tile, core_stride):
  D = table0.shape[1]
  nchunk = per_tile // CH
  ng = nchunk // 8

  @functools.partial(
      pl.kernel,
      out_type=jax.ShapeDtypeStruct((NC, NP, D), jnp.float32),
      mesh=_MESH,
      compiler_params=pltpu.CompilerParams(use_tc_tiling_on_sc=False),
      scratch_types=(
          [pltpu.VMEM((CH,), jnp.int32) for _ in range(16)]
          + [pltpu.VMEM((CH, D), jnp.float32) for _ in range(8)]
          + [pltpu.VMEM_SHARED((NP, D), jnp.float32),
             pltpu.VMEM_SHARED((NP, D), jnp.float32)]
          + [pltpu.SemaphoreType.DMA for _ in range(28)]
      ),
  )
  def k(t0_hbm, t1_hbm, src_hbm, dst_hbm, zeros_hbm, out_hbm, *rest):
    srcb = rest[0:8]
    dstb = rest[8:16]
    rows = rest[16:24]
    acc = rest[24]
    tb = rest[25]
    sis = rest[26:34]
    sid = rest[34:42]
    sg = rest[42:50]
    ssc = rest[50:54]
    ci = lax.axis_index("c")
    si = lax.axis_index("s")
    sl = pl.ds(si * TPR, TPR)
    pltpu.sync_copy(zeros_hbm, acc.at[sl])

    @pl.when(ci == 0)
    def _():
      pltpu.sync_copy(t0_hbm.at[sl], tb.at[sl])

    @pl.when(ci == 1)
    def _():
      pltpu.sync_copy(t1_hbm.at[sl], tb.at[sl])

    plsc.subcore_barrier()

    def emit(cbase):
      def src_dma(kk, b):
        pltpu.async_copy(src_hbm.at[cbase + kk], srcb[b], sis[b])

      def dst_dma(kk, b):
        pltpu.async_copy(dst_hbm.at[cbase + kk], dstb[b], sid[b])

      def gather(kk_b):
        pltpu.make_async_copy(src_hbm.at[0], srcb[kk_b], sis[kk_b]).wait()
        pltpu.async_copy(tb.at[srcb[kk_b]], rows[kk_b], sg[kk_b])

      def scatter_wait(b4):
        pltpu.make_async_copy(rows[0], acc.at[dstb[0]], ssc[b4]).wait()

      for b in range(8):
        src_dma(b, b)
      for b in range(4):
        dst_dma(b, b)
      for b in range(4):
        gather(b)

      def body(g, carry):
        for b in range(8):
          kk = g * 8 + b
          b4 = b % 4
          bn = (b + 4) % 8
          pltpu.make_async_copy(tb.at[srcb[b]], rows[b], sg[b]).wait()
          pltpu.make_async_copy(src_hbm.at[0], dstb[b], sid[b]).wait()
          if b < 4:
            pl.when(g > 0)(lambda: scatter_wait(b4))
          else:
            scatter_wait(b4)
          pltpu.async_copy(rows[b], acc.at[dstb[b]], ssc[b4], add=True)

          def refill(kk=kk, b=b, bn=bn):
            dst_dma(kk + 4, bn)
            gather(bn)
          if b < 4:
            refill()
          else:
            pl.when(g < ng - 1)(refill)
          pl.when(g < ng - 1)(lambda kk=kk, b=b: src_dma(kk + 8, b))
        return carry

      lax.fori_loop(0, ng, body, 0)
      for i in range(4):
        scatter_wait(i)

    emit(si * (per_tile // CH) + ci * (core_stride // CH))
    plsc.subcore_barrier()
    pltpu.sync_copy(acc.at[pl.ds(si * TPR, TPR)],
                    out_hbm.at[ci, pl.ds(si * TPR, TPR)])

  return k(table0, table1, src2d, dst2d, zeros)


def _sc_deg(dst2d, ones, zeros):
  D = ones.shape[1]
  chunk = dst2d.shape[1]
  nchunk = PER_TILE // chunk

  @functools.partial(
      pl.kernel,
      out_type=jax.ShapeDtypeStruct((NC, NP, D), jnp.float32),
      mesh=_MESH,
      compiler_params=pltpu.CompilerParams(use_tc_tiling_on_sc=False),
      scratch_types=(
          [pltpu.VMEM((nchunk, chunk), jnp.int32),
           pltpu.VMEM((chunk, D), jnp.float32),
           pltpu.VMEM_SHARED((NP, D), jnp.float32)]
          + [pltpu.SemaphoreType.DMA for _ in range(5)]
      ),
  )
  def k(dst_hbm, ones_hbm, zeros_hbm, out_hbm, dstb, onev, acc, *ss):
    ci = lax.axis_index("c")
    si = lax.axis_index("s")
    cbase = (ci * NS + si) * nchunk
    pltpu.sync_copy(zeros_hbm, acc.at[pl.ds(si * TPR, TPR)])
    pltpu.sync_copy(ones_hbm, onev)
    pltpu.sync_copy(dst_hbm.at[pl.ds(cbase, nchunk)], dstb)
    plsc.subcore_barrier()

    def scatter(kk, b5):
      pltpu.async_copy(onev, acc.at[dstb.at[kk]], ss[b5], add=True)

    def scatter_wait(kk, b5):
      pltpu.make_async_copy(onev, acc.at[dstb.at[kk]], ss[b5]).wait()

    for b in range(5):
      scatter(b, b)

    def body(g, carry):
      for b in range(5):
        kk = 5 + g * 5 + b
        scatter_wait(kk - 5, b)
        scatter(kk, b)
      return carry

    lax.fori_loop(0, (nchunk - 5) // 5, body, 0)
    for i in range(5):
      kk = nchunk - 5 + i
      scatter_wait(kk, i)
    plsc.subcore_barrier()
    pltpu.sync_copy(acc.at[pl.ds(si * TPR, TPR)],
                    out_hbm.at[ci, pl.ds(si * TPR, TPR)])

  return k(dst2d, ones, zeros)


_R = 1024


def _tc_t1(x, W1, degp):
  def body(x_ref, w_ref, dp_ref, t1_ref, deg_ref):
    deg = dp_ref[0] + dp_ref[1] + 1.0
    dinv = lax.rsqrt(deg[:, :1])
    t1_ref[...] = dinv * jnp.dot(x_ref[...], w_ref[...],
                                 preferred_element_type=jnp.float32)
    deg_ref[...] = deg

  return pl.pallas_call(
      body,
      grid=(NP // _R,),
      in_specs=[
          pl.BlockSpec((_R, x.shape[1]), lambda i: (i, 0)),
          pl.BlockSpec(W1.shape, lambda i: (0, 0)),
          pl.BlockSpec((NC, _R, degp.shape[2]), lambda i: (0, i, 0)),
      ],
      out_specs=[
          pl.BlockSpec((_R, W1.shape[1]), lambda i: (i, 0)),
          pl.BlockSpec((_R, degp.shape[2]), lambda i: (i, 0)),
      ],
      out_shape=[
          jax.ShapeDtypeStruct((NP, W1.shape[1]), jnp.float32),
          jax.ShapeDtypeStruct((NP, degp.shape[2]), jnp.float32),
      ],
  )(x, W1, degp)


def _tc_t2(accp, T1, deg, b1, W2):
  def body(a_ref, t1_ref, deg_ref, b_ref, w_ref, t2a_ref, t2b_ref):
    dinv = lax.rsqrt(deg_ref[:, :1])
    h = jnp.maximum(dinv * (a_ref[0] + a_ref[1] + t1_ref[...]) + b_ref[...],
                    0.0)
    t2 = dinv * jnp.dot(h, w_ref[...], preferred_element_type=jnp.float32)
    half = w_ref.shape[1] // 2
    t2a_ref[...] = t2[:, :half]
    t2b_ref[...] = t2[:, half:]

  D = T1.shape[1]
  half = W2.shape[1] // 2
  return pl.pallas_call(
      body,
      grid=(NP // _R,),
      in_specs=[
          pl.BlockSpec((NC, _R, D), lambda i: (0, i, 0)),
          pl.BlockSpec((_R, D), lambda i: (i, 0)),
          pl.BlockSpec((_R, deg.shape[1]), lambda i: (i, 0)),
          pl.BlockSpec(b1.shape, lambda i: (0, 0)),
          pl.BlockSpec(W2.shape, lambda i: (0, 0)),
      ],
      out_specs=[
          pl.BlockSpec((_R, half), lambda i: (i, 0)),
          pl.BlockSpec((_R, half), lambda i: (i, 0)),
      ],
      out_shape=[
          jax.ShapeDtypeStruct((NP, half), jnp.float32),
          jax.ShapeDtypeStruct((NP, half), jnp.float32),
      ],
  )(accp, T1, deg, b1, W2)


def _tc_out(accp, T2a, T2b, deg, b2):
  def body(a_ref, ta_ref, tb_ref, deg_ref, b_ref, o_ref):
    dinv = lax.rsqrt(deg_ref[:, :1])
    t = jnp.concatenate([a_ref[0] + ta_ref[...], a_ref[1] + tb_ref[...]],
                        axis=1)
    o_ref[...] = dinv * t + b_ref[...]

  half = T2a.shape[1]
  return pl.pallas_call(
      body,
      grid=(NP // _R,),
      in_specs=[
          pl.BlockSpec((NC, _R, half), lambda i: (0, i, 0)),
          pl.BlockSpec((_R, half), lambda i: (i, 0)),
          pl.BlockSpec((_R, half), lambda i: (i, 0)),
          pl.BlockSpec((_R, deg.shape[1]), lambda i: (i, 0)),
          pl.BlockSpec(b2.shape, lambda i: (0, 0)),
      ],
      out_specs=pl.BlockSpec((_R, 2 * half), lambda i: (i, 0)),
      out_shape=jax.ShapeDtypeStruct((NP, 2 * half), jnp.float32),
  )(accp, T2a, T2b, deg, b2)


def kernel(x, edge_index, W1, b1, W2, b2):
  pad = jnp.full((E_PAD - E,), N, jnp.int32)
  src128 = jnp.concatenate([edge_index[0], pad]).reshape(-1, CH)
  dst128 = jnp.concatenate([edge_index[1], pad]).reshape(-1, CH)
  x = jnp.pad(x, ((0, NP - N), (0, 0)))
  ones8 = jnp.ones((CH, 8), jnp.float32)
  z8 = jnp.zeros((TPR, 8), jnp.float32)
  z64 = jnp.zeros((TPR, 64), jnp.float32)

  degp = _sc_deg(dst128, ones8, z8)
  T1, deg = _tc_t1(x, W1, degp)
  acc1 = _sc_agg(T1, T1, src128, dst128, z64, PER_TILE, E_PAD // NC)
  T2a, T2b = _tc_t2(acc1, T1, deg, b1.reshape(1, -1), W2)
  acc2 = _sc_agg(T2a, T2b, src128, dst128, z64, E_PAD // NS, 0)
  out = _tc_out(acc2, T2a, T2b, deg, b2.reshape(1, -1))
  return out[:N]


# speedup vs baseline: 41.5885x; 1.3130x over previous
import functools

import jax
import jax.numpy as jnp
from jax import lax
from jax.experimental import pallas as pl
from jax.experimental.pallas import tpu as pltpu
from jax.experimental.pallas import tpu_sc as plsc

N = 10000
NP = 10240
E = 320000
NC = 2
NS = 16
TPR = NP // NS
PER_TILE = 10240
E_PAD = NC * NS * PER_TILE
CH = 128

_MESH = plsc.VectorSubcoreMesh(
    core_axis_name="c", subcore_axis_name="s", num_cores=NC, num_subcores=NS)


def _sc_agg(table0, table1, src2d, dst2d, zeros, per_tile, core_stride):
  D = table0.shape[1]
  nchunk = per_tile // CH
  ng = nchunk // 8

  @functools.partial(
      pl.kernel,
      out_type=jax.ShapeDtypeStruct((NC, NP, D), jnp.bfloat16),
      mesh=_MESH,
      compiler_params=pltpu.CompilerParams(use_tc_tiling_on_sc=False),
      scratch_types=(
          [pltpu.VMEM((CH,), jnp.int32) for _ in range(16)]
          + [pltpu.VMEM((CH, D), jnp.bfloat16) for _ in range(8)]
          + [pltpu.VMEM_SHARED((NP, D), jnp.bfloat16),
             pltpu.VMEM_SHARED((NP, D), jnp.bfloat16)]
          + [pltpu.SemaphoreType.DMA for _ in range(28)]
      ),
  )
  def k(t0_hbm, t1_hbm, src_hbm, dst_hbm, zeros_hbm, out_hbm, *rest):
    srcb = rest[0:8]
    dstb = rest[8:16]
    rows = rest[16:24]
    acc = rest[24]
    tb = rest[25]
    sis = rest[26:34]
    sid = rest[34:42]
    sg = rest[42:50]
    ssc = rest[50:54]
    ci = lax.axis_index("c")
    si = lax.axis_index("s")
    sl = pl.ds(si * TPR, TPR)
    pltpu.sync_copy(zeros_hbm, acc.at[sl])

    @pl.when(ci == 0)
    def _():
      pltpu.sync_copy(t0_hbm.at[sl], tb.at[sl])

    @pl.when(ci == 1)
    def _():
      pltpu.sync_copy(t1_hbm.at[sl], tb.at[sl])

    plsc.subcore_barrier()

    def emit(cbase):
      def src_dma(kk, b):
        pltpu.async_copy(src_hbm.at[cbase + kk], srcb[b], sis[b])

      def dst_dma(kk, b):
        pltpu.async_copy(dst_hbm.at[cbase + kk], dstb[b], sid[b])

      def gather(kk_b):
        pltpu.make_async_copy(src_hbm.at[0], srcb[kk_b], sis[kk_b]).wait()
        pltpu.async_copy(tb.at[srcb[kk_b]], rows[kk_b], sg[kk_b])

      def scatter_wait(b4):
        pltpu.make_async_copy(rows[0], acc.at[dstb[0]], ssc[b4]).wait()

      for b in range(8):
        src_dma(b, b)
      for b in range(4):
        dst_dma(b, b)
      for b in range(4):
        gather(b)

      def body(g, carry):
        for b in range(8):
          kk = g * 8 + b
          b4 = b % 4
          bn = (b + 4) % 8
          pltpu.make_async_copy(tb.at[srcb[b]], rows[b], sg[b]).wait()
          pltpu.make_async_copy(src_hbm.at[0], dstb[b], sid[b]).wait()
          if b < 4:
            pl.when(g > 0)(lambda: scatter_wait(b4))
          else:
            scatter_wait(b4)
          pltpu.async_copy(rows[b], acc.at[dstb[b]], ssc[b4], add=True)

          def refill(kk=kk, b=b, bn=bn):
            dst_dma(kk + 4, bn)
            gather(bn)
          if b < 4:
            refill()
          else:
            pl.when(g < ng - 1)(refill)
          pl.when(g < ng - 1)(lambda kk=kk, b=b: src_dma(kk + 8, b))
        return carry

      lax.fori_loop(0, ng, body, 0)
      for i in range(4):
        scatter_wait(i)

    emit(si * (per_tile // CH) + ci * (core_stride // CH))
    plsc.subcore_barrier()
    pltpu.sync_copy(acc.at[pl.ds(si * TPR, TPR)],
                    out_hbm.at[ci, pl.ds(si * TPR, TPR)])

  return k(table0, table1, src2d, dst2d, zeros)


def _sc_deg(dst2d, ones, zeros):
  D = ones.shape[1]
  chunk = dst2d.shape[1]
  nchunk = PER_TILE // chunk

  @functools.partial(
      pl.kernel,
      out_type=jax.ShapeDtypeStruct((NC, NP, D), jnp.float32),
      mesh=_MESH,
      compiler_params=pltpu.CompilerParams(use_tc_tiling_on_sc=False),
      scratch_types=(
          [pltpu.VMEM((nchunk, chunk), jnp.int32),
           pltpu.VMEM((chunk, D), jnp.float32),
           pltpu.VMEM_SHARED((NP, D), jnp.float32)]
          + [pltpu.SemaphoreType.DMA for _ in range(5)]
      ),
  )
  def k(dst_hbm, ones_hbm, zeros_hbm, out_hbm, dstb, onev, acc, *ss):
    ci = lax.axis_index("c")
    si = lax.axis_index("s")
    cbase = (ci * NS + si) * nchunk
    pltpu.sync_copy(zeros_hbm, acc.at[pl.ds(si * TPR, TPR)])
    pltpu.sync_copy(ones_hbm, onev)
    pltpu.sync_copy(dst_hbm.at[pl.ds(cbase, nchunk)], dstb)
    plsc.subcore_barrier()

    def scatter(kk, b5):
      pltpu.async_copy(onev, acc.at[dstb.at[kk]], ss[b5], add=True)

    def scatter_wait(kk, b5):
      pltpu.make_async_copy(onev, acc.at[dstb.at[kk]], ss[b5]).wait()

    for b in range(5):
      scatter(b, b)

    def body(g, carry):
      for b in range(5):
        kk = 5 + g * 5 + b
        scatter_wait(kk - 5, b)
        scatter(kk, b)
      return carry

    lax.fori_loop(0, (nchunk - 5) // 5, body, 0)
    for i in range(5):
      kk = nchunk - 5 + i
      scatter_wait(kk, i)
    plsc.subcore_barrier()
    pltpu.sync_copy(acc.at[pl.ds(si * TPR, TPR)],
                    out_hbm.at[ci, pl.ds(si * TPR, TPR)])

  return k(dst2d, ones, zeros)


_R = 1024


def _tc_t1(x, W1, degp):
  def body(x_ref, w_ref, dp_ref, t1_ref, deg_ref):
    deg = dp_ref[0] + dp_ref[1] + 1.0
    dinv = lax.rsqrt(deg[:, :1])
    t1_ref[...] = (dinv * jnp.dot(x_ref[...], w_ref[...],
                                  preferred_element_type=jnp.float32)
                   ).astype(jnp.bfloat16)
    deg_ref[...] = deg

  return pl.pallas_call(
      body,
      grid=(NP // _R,),
      in_specs=[
          pl.BlockSpec((_R, x.shape[1]), lambda i: (i, 0)),
          pl.BlockSpec(W1.shape, lambda i: (0, 0)),
          pl.BlockSpec((NC, _R, degp.shape[2]), lambda i: (0, i, 0)),
      ],
      out_specs=[
          pl.BlockSpec((_R, W1.shape[1]), lambda i: (i, 0)),
          pl.BlockSpec((_R, degp.shape[2]), lambda i: (i, 0)),
      ],
      out_shape=[
          jax.ShapeDtypeStruct((NP, W1.shape[1]), jnp.bfloat16),
          jax.ShapeDtypeStruct((NP, degp.shape[2]), jnp.float32),
      ],
  )(x, W1, degp)


def _tc_t2(accp, T1, deg, b1, W2):
  def body(a_ref, t1_ref, deg_ref, b_ref, w_ref, t2a_ref, t2b_ref):
    dinv = lax.rsqrt(deg_ref[:, :1])
    agg = (a_ref[0] + a_ref[1]).astype(jnp.float32) + t1_ref[...].astype(
        jnp.float32)
    h = jnp.maximum(dinv * agg + b_ref[...], 0.0)
    t2 = dinv * jnp.dot(h, w_ref[...], preferred_element_type=jnp.float32)
    half = w_ref.shape[1] // 2
    t2a_ref[...] = t2[:, :half].astype(jnp.bfloat16)
    t2b_ref[...] = t2[:, half:].astype(jnp.bfloat16)

  D = T1.shape[1]
  half = W2.shape[1] // 2
  return pl.pallas_call(
      body,
      grid=(NP // _R,),
      in_specs=[
          pl.BlockSpec((NC, _R, D), lambda i: (0, i, 0)),
          pl.BlockSpec((_R, D), lambda i: (i, 0)),
          pl.BlockSpec((_R, deg.shape[1]), lambda i: (i, 0)),
          pl.BlockSpec(b1.shape, lambda i: (0, 0)),
          pl.BlockSpec(W2.shape, lambda i: (0, 0)),
      ],
      out_specs=[
          pl.BlockSpec((_R, half), lambda i: (i, 0)),
          pl.BlockSpec((_R, half), lambda i: (i, 0)),
      ],
      out_shape=[
          jax.ShapeDtypeStruct((NP, half), jnp.bfloat16),
          jax.ShapeDtypeStruct((NP, half), jnp.bfloat16),
      ],
  )(accp, T1, deg, b1, W2)


def _tc_out(accp, T2a, T2b, deg, b2):
  def body(a_ref, ta_ref, tb_ref, deg_ref, b_ref, o_ref):
    dinv = lax.rsqrt(deg_ref[:, :1])
    t = jnp.concatenate(
        [a_ref[0].astype(jnp.float32) + ta_ref[...].astype(jnp.float32),
         a_ref[1].astype(jnp.float32) + tb_ref[...].astype(jnp.float32)],
        axis=1)
    o_ref[...] = dinv * t + b_ref[...]

  half = T2a.shape[1]
  return pl.pallas_call(
      body,
      grid=(NP // _R,),
      in_specs=[
          pl.BlockSpec((NC, _R, half), lambda i: (0, i, 0)),
          pl.BlockSpec((_R, half), lambda i: (i, 0)),
          pl.BlockSpec((_R, half), lambda i: (i, 0)),
          pl.BlockSpec((_R, deg.shape[1]), lambda i: (i, 0)),
          pl.BlockSpec(b2.shape, lambda i: (0, 0)),
      ],
      out_specs=pl.BlockSpec((_R, 2 * half), lambda i: (i, 0)),
      out_shape=jax.ShapeDtypeStruct((NP, 2 * half), jnp.float32),
  )(accp, T2a, T2b, deg, b2)


def kernel(x, edge_index, W1, b1, W2, b2):
  pad = jnp.full((E_PAD - E,), N, jnp.int32)
  src128 = jnp.concatenate([edge_index[0], pad]).reshape(-1, CH)
  dst128 = jnp.concatenate([edge_index[1], pad]).reshape(-1, CH)
  x = jnp.pad(x, ((0, NP - N), (0, 0)))
  ones8 = jnp.ones((CH, 8), jnp.float32)
  z8 = jnp.zeros((TPR, 8), jnp.float32)
  z64 = jnp.zeros((TPR, 64), jnp.bfloat16)

  degp = _sc_deg(dst128, ones8, z8)
  T1, deg = _tc_t1(x, W1, degp)
  acc1 = _sc_agg(T1, T1, src128, dst128, z64, PER_TILE, E_PAD // NC)
  T2a, T2b = _tc_t2(acc1, T1, deg, b1.reshape(1, -1), W2)
  acc2 = _sc_agg(T2a, T2b, src128, dst128, z64, E_PAD // NS, 0)
  out = _tc_out(acc2, T2a, T2b, deg, b2.reshape(1, -1))
  return out[:N]


# speedup vs baseline: 43.7968x; 1.0531x over previous
import functools

import jax
import jax.numpy as jnp
from jax import lax
from jax.experimental import pallas as pl
from jax.experimental.pallas import tpu as pltpu
from jax.experimental.pallas import tpu_sc as plsc

N = 10000
NP = 10240
E = 320000
NC = 2
NS = 16
TPR = NP // NS
PER_TILE = 10240
E_PAD = NC * NS * PER_TILE
CH = 128

_MESH = plsc.VectorSubcoreMesh(
    core_axis_name="c", subcore_axis_name="s", num_cores=NC, num_subcores=NS)


def _sc_agg(table0, table1, src2d, dst2d, zeros, per_tile, core_stride):
  D = table0.shape[1]
  nchunk = per_tile // CH
  ng = nchunk // 8

  @functools.partial(
      pl.kernel,
      out_type=jax.ShapeDtypeStruct((NC, NP, D), jnp.bfloat16),
      mesh=_MESH,
      compiler_params=pltpu.CompilerParams(use_tc_tiling_on_sc=False),
      scratch_types=(
          [pltpu.VMEM((CH,), jnp.int32) for _ in range(16)]
          + [pltpu.VMEM((CH, D), jnp.bfloat16) for _ in range(8)]
          + [pltpu.VMEM_SHARED((NP, D), jnp.bfloat16),
             pltpu.VMEM_SHARED((NP, D), jnp.bfloat16)]
          + [pltpu.SemaphoreType.DMA for _ in range(28)]
      ),
  )
  def k(t0_hbm, t1_hbm, src_hbm, dst_hbm, zeros_hbm, out_hbm, *rest):
    srcb = rest[0:8]
    dstb = rest[8:16]
    rows = rest[16:24]
    acc = rest[24]
    tb = rest[25]
    sis = rest[26:34]
    sid = rest[34:42]
    sg = rest[42:50]
    ssc = rest[50:54]
    ci = lax.axis_index("c")
    si = lax.axis_index("s")
    sl = pl.ds(si * TPR, TPR)
    pltpu.sync_copy(zeros_hbm, acc.at[sl])

    @pl.when(ci == 0)
    def _():
      pltpu.sync_copy(t0_hbm.at[sl], tb.at[sl])

    @pl.when(ci == 1)
    def _():
      pltpu.sync_copy(t1_hbm.at[sl], tb.at[sl])

    plsc.subcore_barrier()

    def emit(cbase):
      def src_dma(kk, b):
        pltpu.async_copy(src_hbm.at[cbase + kk], srcb[b], sis[b])

      def dst_dma(kk, b):
        pltpu.async_copy(dst_hbm.at[cbase + kk], dstb[b], sid[b])

      def gather(kk_b):
        pltpu.make_async_copy(src_hbm.at[0], srcb[kk_b], sis[kk_b]).wait()
        pltpu.async_copy(tb.at[srcb[kk_b]], rows[kk_b], sg[kk_b])

      def scatter_wait(b4):
        pltpu.make_async_copy(rows[0], acc.at[dstb[0]], ssc[b4]).wait()

      for b in range(8):
        src_dma(b, b)
      for b in range(4):
        dst_dma(b, b)
      for b in range(4):
        gather(b)

      def body(g, carry):
        for b in range(8):
          kk = g * 8 + b
          b4 = b % 4
          bn = (b + 4) % 8
          pltpu.make_async_copy(tb.at[srcb[b]], rows[b], sg[b]).wait()
          pltpu.make_async_copy(src_hbm.at[0], dstb[b], sid[b]).wait()
          if b < 4:
            pl.when(g > 0)(lambda: scatter_wait(b4))
          else:
            scatter_wait(b4)
          pltpu.async_copy(rows[b], acc.at[dstb[b]], ssc[b4], add=True)

          def refill(kk=kk, b=b, bn=bn):
            dst_dma(kk + 4, bn)
            gather(bn)
          if b < 4:
            refill()
          else:
            pl.when(g < ng - 1)(refill)
          pl.when(g < ng - 1)(lambda kk=kk, b=b: src_dma(kk + 8, b))
        return carry

      lax.fori_loop(0, ng, body, 0)
      for i in range(4):
        scatter_wait(i)

    emit(si * (per_tile // CH) + ci * (core_stride // CH))
    plsc.subcore_barrier()
    pltpu.sync_copy(acc.at[pl.ds(si * TPR, TPR)],
                    out_hbm.at[ci, pl.ds(si * TPR, TPR)])

  return k(table0, table1, src2d, dst2d, zeros)


def _sc_deg(dst2d, ones, zeros):
  D = ones.shape[1]
  chunk = dst2d.shape[1]
  nchunk = PER_TILE // chunk

  @functools.partial(
      pl.kernel,
      out_type=jax.ShapeDtypeStruct((NC, NP, D), jnp.float32),
      mesh=_MESH,
      compiler_params=pltpu.CompilerParams(use_tc_tiling_on_sc=False),
      scratch_types=(
          [pltpu.VMEM((nchunk, chunk), jnp.int32),
           pltpu.VMEM((chunk, D), jnp.float32),
           pltpu.VMEM_SHARED((NP, D), jnp.float32)]
          + [pltpu.SemaphoreType.DMA for _ in range(5)]
      ),
  )
  def k(dst_hbm, ones_hbm, zeros_hbm, out_hbm, dstb, onev, acc, *ss):
    ci = lax.axis_index("c")
    si = lax.axis_index("s")
    cbase = (ci * NS + si) * nchunk
    pltpu.sync_copy(zeros_hbm, acc.at[pl.ds(si * TPR, TPR)])
    pltpu.sync_copy(ones_hbm, onev)
    pltpu.sync_copy(dst_hbm.at[pl.ds(cbase, nchunk)], dstb)
    plsc.subcore_barrier()

    def scatter(kk, b5):
      pltpu.async_copy(onev, acc.at[dstb.at[kk]], ss[b5], add=True)

    def scatter_wait(kk, b5):
      pltpu.make_async_copy(onev, acc.at[dstb.at[kk]], ss[b5]).wait()

    for b in range(5):
      scatter(b, b)

    def body(g, carry):
      for b in range(5):
        kk = 5 + g * 5 + b
        scatter_wait(kk - 5, b)
        scatter(kk, b)
      return carry

    lax.fori_loop(0, (nchunk - 5) // 5, body, 0)
    for i in range(5):
      kk = nchunk - 5 + i
      scatter_wait(kk, i)
    plsc.subcore_barrier()
    pltpu.sync_copy(acc.at[pl.ds(si * TPR, TPR)],
                    out_hbm.at[ci, pl.ds(si * TPR, TPR)])

  return k(dst2d, ones, zeros)


_R = 1024


def _tc_xw1(x, W1):
  def body(x_ref, w_ref, o_ref):
    o_ref[...] = jnp.dot(x_ref[...], w_ref[...],
                         preferred_element_type=jnp.float32)

  return pl.pallas_call(
      body,
      grid=(NP // _R,),
      in_specs=[
          pl.BlockSpec((_R, x.shape[1]), lambda i: (i, 0)),
          pl.BlockSpec(W1.shape, lambda i: (0, 0)),
      ],
      out_specs=pl.BlockSpec((_R, W1.shape[1]), lambda i: (i, 0)),
      out_shape=jax.ShapeDtypeStruct((NP, W1.shape[1]), jnp.float32),
  )(x, W1)


def _tc_t1(xw1, degp):
  def body(xw_ref, dp_ref, t1_ref, deg_ref):
    deg = dp_ref[0] + dp_ref[1] + 1.0
    dinv = lax.rsqrt(deg[:, :1])
    t1_ref[...] = (dinv * xw_ref[...]).astype(jnp.bfloat16)
    deg_ref[...] = deg

  return pl.pallas_call(
      body,
      grid=(NP // _R,),
      in_specs=[
          pl.BlockSpec((_R, xw1.shape[1]), lambda i: (i, 0)),
          pl.BlockSpec((NC, _R, degp.shape[2]), lambda i: (0, i, 0)),
      ],
      out_specs=[
          pl.BlockSpec((_R, xw1.shape[1]), lambda i: (i, 0)),
          pl.BlockSpec((_R, degp.shape[2]), lambda i: (i, 0)),
      ],
      out_shape=[
          jax.ShapeDtypeStruct((NP, xw1.shape[1]), jnp.bfloat16),
          jax.ShapeDtypeStruct((NP, degp.shape[2]), jnp.float32),
      ],
  )(xw1, degp)


def _tc_t2(accp, T1, deg, b1, W2):
  def body(a_ref, t1_ref, deg_ref, b_ref, w_ref, t2a_ref, t2b_ref):
    dinv = lax.rsqrt(deg_ref[:, :1])
    agg = (a_ref[0] + a_ref[1]).astype(jnp.float32) + t1_ref[...].astype(
        jnp.float32)
    h = jnp.maximum(dinv * agg + b_ref[...], 0.0)
    t2 = dinv * jnp.dot(h, w_ref[...], preferred_element_type=jnp.float32)
    half = w_ref.shape[1] // 2
    t2a_ref[...] = t2[:, :half].astype(jnp.bfloat16)
    t2b_ref[...] = t2[:, half:].astype(jnp.bfloat16)

  D = T1.shape[1]
  half = W2.shape[1] // 2
  return pl.pallas_call(
      body,
      grid=(NP // _R,),
      in_specs=[
          pl.BlockSpec((NC, _R, D), lambda i: (0, i, 0)),
          pl.BlockSpec((_R, D), lambda i: (i, 0)),
          pl.BlockSpec((_R, deg.shape[1]), lambda i: (i, 0)),
          pl.BlockSpec(b1.shape, lambda i: (0, 0)),
          pl.BlockSpec(W2.shape, lambda i: (0, 0)),
      ],
      out_specs=[
          pl.BlockSpec((_R, half), lambda i: (i, 0)),
          pl.BlockSpec((_R, half), lambda i: (i, 0)),
      ],
      out_shape=[
          jax.ShapeDtypeStruct((NP, half), jnp.bfloat16),
          jax.ShapeDtypeStruct((NP, half), jnp.bfloat16),
      ],
  )(accp, T1, deg, b1, W2)


def _tc_out(accp, T2a, T2b, deg, b2):
  def body(a_ref, ta_ref, tb_ref, deg_ref, b_ref, o_ref):
    dinv = lax.rsqrt(deg_ref[:, :1])
    t = jnp.concatenate(
        [a_ref[0].astype(jnp.float32) + ta_ref[...].astype(jnp.float32),
         a_ref[1].astype(jnp.float32) + tb_ref[...].astype(jnp.float32)],
        axis=1)
    o_ref[...] = dinv * t + b_ref[...]

  half = T2a.shape[1]
  return pl.pallas_call(
      body,
      grid=(NP // _R,),
      in_specs=[
          pl.BlockSpec((NC, _R, half), lambda i: (0, i, 0)),
          pl.BlockSpec((_R, half), lambda i: (i, 0)),
          pl.BlockSpec((_R, half), lambda i: (i, 0)),
          pl.BlockSpec((_R, deg.shape[1]), lambda i: (i, 0)),
          pl.BlockSpec(b2.shape, lambda i: (0, 0)),
      ],
      out_specs=pl.BlockSpec((_R, 2 * half), lambda i: (i, 0)),
      out_shape=jax.ShapeDtypeStruct((NP, 2 * half), jnp.float32),
  )(accp, T2a, T2b, deg, b2)


def kernel(x, edge_index, W1, b1, W2, b2):
  pads = N + (jnp.arange(E_PAD - E, dtype=jnp.int32) % (NP - N))
  src128 = jnp.concatenate([edge_index[0], pads]).reshape(-1, CH)
  dst128 = jnp.concatenate([edge_index[1], pads]).reshape(-1, CH)
  x = jnp.pad(x, ((0, NP - N), (0, 0)))
  ones8 = jnp.ones((CH, 8), jnp.float32)
  z8 = jnp.zeros((TPR, 8), jnp.float32)
  z64 = jnp.zeros((TPR, 64), jnp.bfloat16)

  xw1 = _tc_xw1(x, W1)
  degp = _sc_deg(dst128, ones8, z8)
  T1, deg = _tc_t1(xw1, degp)
  acc1 = _sc_agg(T1, T1, src128, dst128, z64, PER_TILE, E_PAD // NC)
  T2a, T2b = _tc_t2(acc1, T1, deg, b1.reshape(1, -1), W2)
  acc2 = _sc_agg(T2a, T2b, src128, dst128, z64, E_PAD // NS, 0)
  out = _tc_out(acc2, T2a, T2b, deg, b2.reshape(1, -1))
  return out[:N]


# speedup vs baseline: 45.0401x; 1.0284x over previous
import functools

import jax
import jax.numpy as jnp
from jax import lax
from jax.experimental import pallas as pl
from jax.experimental.pallas import tpu as pltpu
from jax.experimental.pallas import tpu_sc as plsc

N = 10000
NP = 10240
E = 320000
NC = 2
NS = 16
TPR = NP // NS
PER_TILE = 10240
E_PAD = NC * NS * PER_TILE
CH = 128

_MESH = plsc.VectorSubcoreMesh(
    core_axis_name="c", subcore_axis_name="s", num_cores=NC, num_subcores=NS)


def _sc_agg(table0, table1, edges, zeros, per_tile, core_stride):
  D = table0.shape[1]
  nchunk = per_tile // CH
  ng = nchunk // 8

  @functools.partial(
      pl.kernel,
      out_type=jax.ShapeDtypeStruct((NC, NP, D), jnp.bfloat16),
      mesh=_MESH,
      compiler_params=pltpu.CompilerParams(use_tc_tiling_on_sc=False),
      scratch_types=(
          [pltpu.VMEM((CH,), jnp.int32) for _ in range(16)]
          + [pltpu.VMEM((CH, D), jnp.bfloat16) for _ in range(8)]
          + [pltpu.VMEM_SHARED((NP, D), jnp.bfloat16),
             pltpu.VMEM_SHARED((NP, D), jnp.bfloat16)]
          + [pltpu.SemaphoreType.DMA for _ in range(28)]
      ),
  )
  def k(t0_hbm, t1_hbm, e_hbm, zeros_hbm, out_hbm, *rest):
    srcb = rest[0:8]
    dstb = rest[8:16]
    rows = rest[16:24]
    acc = rest[24]
    tb = rest[25]
    sis = rest[26:34]
    sid = rest[34:42]
    sg = rest[42:50]
    ssc = rest[50:54]
    ci = lax.axis_index("c")
    si = lax.axis_index("s")
    sl = pl.ds(si * TPR, TPR)
    pltpu.sync_copy(zeros_hbm, acc.at[sl])

    @pl.when(ci == 0)
    def _():
      pltpu.sync_copy(t0_hbm.at[sl], tb.at[sl])

    @pl.when(ci == 1)
    def _():
      pltpu.sync_copy(t1_hbm.at[sl], tb.at[sl])

    plsc.subcore_barrier()

    def emit(cbase):
      def src_dma(kk, b):
        pltpu.async_copy(e_hbm.at[0, cbase + kk], srcb[b], sis[b])

      def dst_dma(kk, b):
        pltpu.async_copy(e_hbm.at[1, cbase + kk], dstb[b], sid[b])

      def gather(kk_b):
        pltpu.make_async_copy(e_hbm.at[0, 0], srcb[kk_b], sis[kk_b]).wait()
        pltpu.async_copy(tb.at[srcb[kk_b]], rows[kk_b], sg[kk_b])

      def scatter_wait(b4):
        pltpu.make_async_copy(rows[0], acc.at[dstb[0]], ssc[b4]).wait()

      for b in range(8):
        src_dma(b, b)
      for b in range(4):
        dst_dma(b, b)
      for b in range(4):
        gather(b)

      def body(g, carry):
        for b in range(8):
          kk = g * 8 + b
          b4 = b % 4
          bn = (b + 4) % 8
          pltpu.make_async_copy(tb.at[srcb[b]], rows[b], sg[b]).wait()
          pltpu.make_async_copy(e_hbm.at[0, 0], dstb[b], sid[b]).wait()
          if b < 4:
            pl.when(g > 0)(lambda: scatter_wait(b4))
          else:
            scatter_wait(b4)
          pltpu.async_copy(rows[b], acc.at[dstb[b]], ssc[b4], add=True)

          def refill(kk=kk, b=b, bn=bn):
            dst_dma(kk + 4, bn)
            gather(bn)
          if b < 4:
            refill()
          else:
            pl.when(g < ng - 1)(refill)
          pl.when(g < ng - 1)(lambda kk=kk, b=b: src_dma(kk + 8, b))
        return carry

      lax.fori_loop(0, ng, body, 0)
      for i in range(4):
        scatter_wait(i)

    emit(si * (per_tile // CH) + ci * (core_stride // CH))
    plsc.subcore_barrier()
    pltpu.sync_copy(acc.at[pl.ds(si * TPR, TPR)],
                    out_hbm.at[ci, pl.ds(si * TPR, TPR)])

  return k(table0, table1, edges, zeros)


def _sc_deg(edges, ones, zeros):
  D = ones.shape[1]
  chunk = edges.shape[2]
  nchunk = PER_TILE // chunk

  @functools.partial(
      pl.kernel,
      out_type=jax.ShapeDtypeStruct((NC, NP, D), jnp.float32),
      mesh=_MESH,
      compiler_params=pltpu.CompilerParams(use_tc_tiling_on_sc=False),
      scratch_types=(
          [pltpu.VMEM((nchunk, chunk), jnp.int32),
           pltpu.VMEM((chunk, D), jnp.float32),
           pltpu.VMEM_SHARED((NP, D), jnp.float32)]
          + [pltpu.SemaphoreType.DMA for _ in range(5)]
      ),
  )
  def k(e_hbm, ones_hbm, zeros_hbm, out_hbm, dstb, onev, acc, *ss):
    ci = lax.axis_index("c")
    si = lax.axis_index("s")
    cbase = (ci * NS + si) * nchunk
    pltpu.sync_copy(zeros_hbm, acc.at[pl.ds(si * TPR, TPR)])
    pltpu.sync_copy(ones_hbm, onev)
    pltpu.sync_copy(e_hbm.at[1, pl.ds(cbase, nchunk)], dstb)
    plsc.subcore_barrier()

    def scatter(kk, b5):
      pltpu.async_copy(onev, acc.at[dstb.at[kk]], ss[b5], add=True)

    def scatter_wait(kk, b5):
      pltpu.make_async_copy(onev, acc.at[dstb.at[kk]], ss[b5]).wait()

    for b in range(5):
      scatter(b, b)

    def body(g, carry):
      for b in range(5):
        kk = 5 + g * 5 + b
        scatter_wait(kk - 5, b)
        scatter(kk, b)
      return carry

    lax.fori_loop(0, (nchunk - 5) // 5, body, 0)
    for i in range(5):
      kk = nchunk - 5 + i
      scatter_wait(kk, i)
    plsc.subcore_barrier()
    pltpu.sync_copy(acc.at[pl.ds(si * TPR, TPR)],
                    out_hbm.at[ci, pl.ds(si * TPR, TPR)])

  return k(edges, ones, zeros)


_R = 1024


def _tc_xw1(x, W1):
  def body(x_ref, w_ref, o_ref):
    o_ref[...] = jnp.dot(x_ref[...], w_ref[...],
                         preferred_element_type=jnp.float32)

  return pl.pallas_call(
      body,
      grid=(NP // _R,),
      in_specs=[
          pl.BlockSpec((_R, x.shape[1]), lambda i: (i, 0)),
          pl.BlockSpec(W1.shape, lambda i: (0, 0)),
      ],
      out_specs=pl.BlockSpec((_R, W1.shape[1]), lambda i: (i, 0)),
      out_shape=jax.ShapeDtypeStruct((NP, W1.shape[1]), jnp.float32),
  )(x, W1)


def _tc_t1(xw1, degp):
  def body(xw_ref, dp_ref, t1_ref, deg_ref):
    deg = dp_ref[0] + dp_ref[1] + 1.0
    dinv = lax.rsqrt(deg[:, :1])
    t1_ref[...] = (dinv * xw_ref[...]).astype(jnp.bfloat16)
    deg_ref[...] = deg

  return pl.pallas_call(
      body,
      grid=(NP // _R,),
      in_specs=[
          pl.BlockSpec((_R, xw1.shape[1]), lambda i: (i, 0)),
          pl.BlockSpec((NC, _R, degp.shape[2]), lambda i: (0, i, 0)),
      ],
      out_specs=[
          pl.BlockSpec((_R, xw1.shape[1]), lambda i: (i, 0)),
          pl.BlockSpec((_R, degp.shape[2]), lambda i: (i, 0)),
      ],
      out_shape=[
          jax.ShapeDtypeStruct((NP, xw1.shape[1]), jnp.bfloat16),
          jax.ShapeDtypeStruct((NP, degp.shape[2]), jnp.float32),
      ],
  )(xw1, degp)


def _tc_t2(accp, T1, deg, b1, W2):
  def body(a_ref, t1_ref, deg_ref, b_ref, w_ref, t2a_ref, t2b_ref):
    dinv = lax.rsqrt(deg_ref[:, :1])
    agg = (a_ref[0] + a_ref[1]).astype(jnp.float32) + t1_ref[...].astype(
        jnp.float32)
    h = jnp.maximum(dinv * agg + b_ref[...], 0.0)
    t2 = dinv * jnp.dot(h, w_ref[...], preferred_element_type=jnp.float32)
    half = w_ref.shape[1] // 2
    t2a_ref[...] = t2[:, :half].astype(jnp.bfloat16)
    t2b_ref[...] = t2[:, half:].astype(jnp.bfloat16)

  D = T1.shape[1]
  half = W2.shape[1] // 2
  return pl.pallas_call(
      body,
      grid=(NP // _R,),
      in_specs=[
          pl.BlockSpec((NC, _R, D), lambda i: (0, i, 0)),
          pl.BlockSpec((_R, D), lambda i: (i, 0)),
          pl.BlockSpec((_R, deg.shape[1]), lambda i: (i, 0)),
          pl.BlockSpec(b1.shape, lambda i: (0, 0)),
          pl.BlockSpec(W2.shape, lambda i: (0, 0)),
      ],
      out_specs=[
          pl.BlockSpec((_R, half), lambda i: (i, 0)),
          pl.BlockSpec((_R, half), lambda i: (i, 0)),
      ],
      out_shape=[
          jax.ShapeDtypeStruct((NP, half), jnp.bfloat16),
          jax.ShapeDtypeStruct((NP, half), jnp.bfloat16),
      ],
  )(accp, T1, deg, b1, W2)


def _tc_out(accp, T2a, T2b, deg, b2):
  def body(a_ref, ta_ref, tb_ref, deg_ref, b_ref, o_ref):
    dinv = lax.rsqrt(deg_ref[:, :1])
    t = jnp.concatenate(
        [a_ref[0].astype(jnp.float32) + ta_ref[...].astype(jnp.float32),
         a_ref[1].astype(jnp.float32) + tb_ref[...].astype(jnp.float32)],
        axis=1)
    o_ref[...] = dinv * t + b_ref[...]

  half = T2a.shape[1]
  return pl.pallas_call(
      body,
      grid=(NP // _R,),
      in_specs=[
          pl.BlockSpec((NC, _R, half), lambda i: (0, i, 0)),
          pl.BlockSpec((_R, half), lambda i: (i, 0)),
          pl.BlockSpec((_R, half), lambda i: (i, 0)),
          pl.BlockSpec((_R, deg.shape[1]), lambda i: (i, 0)),
          pl.BlockSpec(b2.shape, lambda i: (0, 0)),
      ],
      out_specs=pl.BlockSpec((_R, 2 * half), lambda i: (i, 0)),
      out_shape=jax.ShapeDtypeStruct((NP, 2 * half), jnp.float32),
  )(accp, T2a, T2b, deg, b2)


def kernel(x, edge_index, W1, b1, W2, b2):
  pads = N + (jnp.arange(E_PAD - E, dtype=jnp.int32) % (NP - N))
  edges = jnp.concatenate(
      [edge_index, jnp.broadcast_to(pads, (2, E_PAD - E))], axis=1
  ).reshape(2, -1, CH)
  x = jnp.pad(x, ((0, NP - N), (0, 0)))
  ones8 = jnp.ones((CH, 8), jnp.float32)
  z8 = jnp.zeros((TPR, 8), jnp.float32)
  z64 = jnp.zeros((TPR, 64), jnp.bfloat16)

  xw1 = _tc_xw1(x, W1)
  degp = _sc_deg(edges, ones8, z8)
  T1, deg = _tc_t1(xw1, degp)
  acc1 = _sc_agg(T1, T1, edges, z64, PER_TILE, E_PAD // NC)
  T2a, T2b = _tc_t2(acc1, T1, deg, b1.reshape(1, -1), W2)
  acc2 = _sc_agg(T2a, T2b, edges, z64, E_PAD // NS, 0)
  out = _tc_out(acc2, T2a, T2b, deg, b2.reshape(1, -1))
  return out[:N]


# speedup vs baseline: 47.1939x; 1.0478x over previous
import functools

import jax
import jax.numpy as jnp
from jax import lax
from jax.experimental import pallas as pl
from jax.experimental.pallas import tpu as pltpu
from jax.experimental.pallas import tpu_sc as plsc

N = 10000
NP = 10240
E = 320000
NC = 2
NS = 16
TPR = NP // NS
PER_TILE = 10240
E_PAD = NC * NS * PER_TILE
CH = 128

_MESH = plsc.VectorSubcoreMesh(
    core_axis_name="c", subcore_axis_name="s", num_cores=NC, num_subcores=NS)


def _sc_agg(table0, table1, edges, zeros, per_tile, core_stride):
  D = table0.shape[1]
  nchunk = per_tile // CH
  ng = nchunk // 8

  @functools.partial(
      pl.kernel,
      out_type=jax.ShapeDtypeStruct((NC, NP, D), jnp.bfloat16),
      mesh=_MESH,
      compiler_params=pltpu.CompilerParams(use_tc_tiling_on_sc=False),
      scratch_types=(
          [pltpu.VMEM((CH,), jnp.int32) for _ in range(16)]
          + [pltpu.VMEM((CH, D), jnp.bfloat16) for _ in range(8)]
          + [pltpu.VMEM_SHARED((NP, D), jnp.bfloat16),
             pltpu.VMEM_SHARED((NP, D), jnp.bfloat16)]
          + [pltpu.SemaphoreType.DMA for _ in range(28)]
      ),
  )
  def k(t0_hbm, t1_hbm, e_hbm, zeros_hbm, out_hbm, *rest):
    srcb = rest[0:8]
    dstb = rest[8:16]
    rows = rest[16:24]
    acc = rest[24]
    tb = rest[25]
    sis = rest[26:34]
    sid = rest[34:42]
    sg = rest[42:50]
    ssc = rest[50:54]
    ci = lax.axis_index("c")
    si = lax.axis_index("s")
    sl = pl.ds(si * TPR, TPR)
    pltpu.sync_copy(zeros_hbm, acc.at[sl])

    @pl.when(ci == 0)
    def _():
      pltpu.sync_copy(t0_hbm.at[sl], tb.at[sl])

    @pl.when(ci == 1)
    def _():
      pltpu.sync_copy(t1_hbm.at[sl], tb.at[sl])

    plsc.subcore_barrier()

    def emit(cbase):
      def src_dma(kk, b):
        pltpu.async_copy(e_hbm.at[0, cbase + kk], srcb[b], sis[b])

      def dst_dma(kk, b):
        pltpu.async_copy(e_hbm.at[1, cbase + kk], dstb[b], sid[b])

      def gather(kk_b):
        pltpu.make_async_copy(e_hbm.at[0, 0], srcb[kk_b], sis[kk_b]).wait()
        pltpu.async_copy(tb.at[srcb[kk_b]], rows[kk_b], sg[kk_b])

      def scatter_wait(b4):
        pltpu.make_async_copy(rows[0], acc.at[dstb[0]], ssc[b4]).wait()

      for b in range(8):
        src_dma(b, b)
      for b in range(4):
        dst_dma(b, b)
      for b in range(4):
        gather(b)

      def body(g, carry):
        for b in range(8):
          kk = g * 8 + b
          b4 = b % 4
          bn = (b + 4) % 8
          pltpu.make_async_copy(tb.at[srcb[b]], rows[b], sg[b]).wait()
          pltpu.make_async_copy(e_hbm.at[0, 0], dstb[b], sid[b]).wait()
          if b < 4:
            pl.when(g > 0)(lambda: scatter_wait(b4))
          else:
            scatter_wait(b4)
          pltpu.async_copy(rows[b], acc.at[dstb[b]], ssc[b4], add=True)

          def refill(kk=kk, b=b, bn=bn):
            dst_dma(kk + 4, bn)
            gather(bn)
          if b < 4:
            refill()
          else:
            pl.when(g < ng - 1)(refill)
          pl.when(g < ng - 1)(lambda kk=kk, b=b: src_dma(kk + 8, b))
        return carry

      lax.fori_loop(0, ng, body, 0)
      for i in range(4):
        scatter_wait(i)

    emit(si * (per_tile // CH) + ci * (core_stride // CH))
    plsc.subcore_barrier()
    pltpu.sync_copy(acc.at[pl.ds(si * TPR, TPR)],
                    out_hbm.at[ci, pl.ds(si * TPR, TPR)])

  return k(table0, table1, edges, zeros)


def _sc_deg(edges, ones, zeros):
  D = ones.shape[1]
  chunk = edges.shape[2]
  nchunk = PER_TILE // chunk

  @functools.partial(
      pl.kernel,
      out_type=jax.ShapeDtypeStruct((NC, NP, D), jnp.float32),
      mesh=_MESH,
      compiler_params=pltpu.CompilerParams(use_tc_tiling_on_sc=False),
      scratch_types=(
          [pltpu.VMEM((nchunk, chunk), jnp.int32),
           pltpu.VMEM((chunk, D), jnp.float32),
           pltpu.VMEM_SHARED((NP, D), jnp.float32)]
          + [pltpu.SemaphoreType.DMA for _ in range(5)]
      ),
  )
  def k(e_hbm, ones_hbm, zeros_hbm, out_hbm, dstb, onev, acc, *ss):
    ci = lax.axis_index("c")
    si = lax.axis_index("s")
    cbase = (ci * NS + si) * nchunk
    pltpu.sync_copy(zeros_hbm, acc.at[pl.ds(si * TPR, TPR)])
    pltpu.sync_copy(ones_hbm, onev)
    pltpu.sync_copy(e_hbm.at[1, pl.ds(cbase, nchunk)], dstb)
    plsc.subcore_barrier()

    def scatter(kk, b5):
      pltpu.async_copy(onev, acc.at[dstb.at[kk]], ss[b5], add=True)

    def scatter_wait(kk, b5):
      pltpu.make_async_copy(onev, acc.at[dstb.at[kk]], ss[b5]).wait()

    for b in range(5):
      scatter(b, b)

    def body(g, carry):
      for b in range(5):
        kk = 5 + g * 5 + b
        scatter_wait(kk - 5, b)
        scatter(kk, b)
      return carry

    lax.fori_loop(0, (nchunk - 5) // 5, body, 0)
    for i in range(5):
      kk = nchunk - 5 + i
      scatter_wait(kk, i)
    plsc.subcore_barrier()
    pltpu.sync_copy(acc.at[pl.ds(si * TPR, TPR)],
                    out_hbm.at[ci, pl.ds(si * TPR, TPR)])

  return k(edges, ones, zeros)


_R = 2048
_RO = 1000


def _tc_xw1(x, W1):
  def body(x_ref, w_ref, o_ref):
    o_ref[...] = jnp.dot(x_ref[...], w_ref[...],
                         preferred_element_type=jnp.float32)

  return pl.pallas_call(
      body,
      grid=(NP // _R,),
      in_specs=[
          pl.BlockSpec((_R, x.shape[1]), lambda i: (i, 0)),
          pl.BlockSpec(W1.shape, lambda i: (0, 0)),
      ],
      out_specs=pl.BlockSpec((_R, W1.shape[1]), lambda i: (i, 0)),
      out_shape=jax.ShapeDtypeStruct((NP, W1.shape[1]), jnp.float32),
  )(x, W1)


def _tc_t1(xw1, degp):
  def body(xw_ref, dp_ref, t1_ref, deg_ref):
    deg = dp_ref[0] + dp_ref[1] + 1.0
    dinv = lax.rsqrt(deg[:, :1])
    t1_ref[...] = (dinv * xw_ref[...]).astype(jnp.bfloat16)
    deg_ref[...] = deg

  return pl.pallas_call(
      body,
      grid=(NP // _R,),
      in_specs=[
          pl.BlockSpec((_R, xw1.shape[1]), lambda i: (i, 0)),
          pl.BlockSpec((NC, _R, degp.shape[2]), lambda i: (0, i, 0)),
      ],
      out_specs=[
          pl.BlockSpec((_R, xw1.shape[1]), lambda i: (i, 0)),
          pl.BlockSpec((_R, degp.shape[2]), lambda i: (i, 0)),
      ],
      out_shape=[
          jax.ShapeDtypeStruct((NP, xw1.shape[1]), jnp.bfloat16),
          jax.ShapeDtypeStruct((NP, degp.shape[2]), jnp.float32),
      ],
  )(xw1, degp)


def _tc_t2(accp, T1, deg, b1, W2):
  def body(a_ref, t1_ref, deg_ref, b_ref, w_ref, t2a_ref, t2b_ref):
    dinv = lax.rsqrt(deg_ref[:, :1])
    agg = (a_ref[0] + a_ref[1]).astype(jnp.float32) + t1_ref[...].astype(
        jnp.float32)
    h = jnp.maximum(dinv * agg + b_ref[...], 0.0)
    t2 = dinv * jnp.dot(h, w_ref[...], preferred_element_type=jnp.float32)
    half = w_ref.shape[1] // 2
    t2a_ref[...] = t2[:, :half].astype(jnp.bfloat16)
    t2b_ref[...] = t2[:, half:].astype(jnp.bfloat16)

  D = T1.shape[1]
  half = W2.shape[1] // 2
  return pl.pallas_call(
      body,
      grid=(NP // _R,),
      in_specs=[
          pl.BlockSpec((NC, _R, D), lambda i: (0, i, 0)),
          pl.BlockSpec((_R, D), lambda i: (i, 0)),
          pl.BlockSpec((_R, deg.shape[1]), lambda i: (i, 0)),
          pl.BlockSpec(b1.shape, lambda i: (0, 0)),
          pl.BlockSpec(W2.shape, lambda i: (0, 0)),
      ],
      out_specs=[
          pl.BlockSpec((_R, half), lambda i: (i, 0)),
          pl.BlockSpec((_R, half), lambda i: (i, 0)),
      ],
      out_shape=[
          jax.ShapeDtypeStruct((NP, half), jnp.bfloat16),
          jax.ShapeDtypeStruct((NP, half), jnp.bfloat16),
      ],
  )(accp, T1, deg, b1, W2)


def _tc_out(accp, T2a, T2b, deg, b2):
  def body(a_ref, ta_ref, tb_ref, deg_ref, b_ref, o_ref):
    dinv = lax.rsqrt(deg_ref[:, :1])
    t = jnp.concatenate(
        [a_ref[0].astype(jnp.float32) + ta_ref[...].astype(jnp.float32),
         a_ref[1].astype(jnp.float32) + tb_ref[...].astype(jnp.float32)],
        axis=1)
    o_ref[...] = dinv * t + b_ref[...]

  half = T2a.shape[1]
  return pl.pallas_call(
      body,
      grid=(N // _RO,),
      in_specs=[
          pl.BlockSpec((NC, _RO, half), lambda i: (0, i, 0)),
          pl.BlockSpec((_RO, half), lambda i: (i, 0)),
          pl.BlockSpec((_RO, half), lambda i: (i, 0)),
          pl.BlockSpec((_RO, deg.shape[1]), lambda i: (i, 0)),
          pl.BlockSpec(b2.shape, lambda i: (0, 0)),
      ],
      out_specs=pl.BlockSpec((_RO, 2 * half), lambda i: (i, 0)),
      out_shape=jax.ShapeDtypeStruct((N, 2 * half), jnp.float32),
  )(accp, T2a, T2b, deg, b2)


def kernel(x, edge_index, W1, b1, W2, b2):
  pads = N + (jnp.arange(E_PAD - E, dtype=jnp.int32) % (NP - N))
  edges = jnp.concatenate(
      [edge_index, jnp.broadcast_to(pads, (2, E_PAD - E))], axis=1
  ).reshape(2, -1, CH)
  x = jnp.pad(x, ((0, NP - N), (0, 0)))
  ones8 = jnp.ones((CH, 8), jnp.float32)
  z8 = jnp.zeros((TPR, 8), jnp.float32)
  z64 = jnp.zeros((TPR, 64), jnp.bfloat16)

  xw1 = _tc_xw1(x, W1)
  degp = _sc_deg(edges, ones8, z8)
  T1, deg = _tc_t1(xw1, degp)
  acc1 = _sc_agg(T1, T1, edges, z64, PER_TILE, E_PAD // NC)
  T2a, T2b = _tc_t2(acc1, T1, deg, b1.reshape(1, -1), W2)
  acc2 = _sc_agg(T2a, T2b, edges, z64, E_PAD // NS, 0)
  return _tc_out(acc2, T2a, T2b, deg, b2.reshape(1, -1))
